# baseline (speedup 1.0000x reference)
import jax
import jax.numpy as jnp
import numpy as np
from jax.experimental import pallas as pl
from jax.experimental.pallas import tpu as pltpu

_B, _N, _INP, _HIDE, _P = 4, 4096, 768, 64, 2
_EPS = 1e-6
_TK = 2048
_T = _B * _N
_NT = _T // _TK
_TPB = _N // _TK

_consts_cache = []


def _build_consts():
    key = jax.random.key(42)
    noise = jax.random.normal(
        jax.random.fold_in(key, 0), (_P, _B, _N, _HIDE), jnp.float32)
    g = jax.random.gumbel(
        jax.random.fold_in(key, 1), (_P, _B * _N * _HIDE, _P), jnp.float32)
    d = (g[..., 0] - g[..., 1]).reshape(_P, _T, _HIDE)
    eps = jax.random.normal(
        jax.random.fold_in(key, 99), (_B, _N, _HIDE),
        jnp.float32).reshape(_T, _HIDE)
    npack = jnp.concatenate(
        [noise[0].reshape(_T, _HIDE), noise[1].reshape(_T, _HIDE)], axis=1)
    dpack = jnp.concatenate([d[0], d[1]], axis=1)
    sh = jnp.asarray(np.block(
        [[np.ones((_HIDE, _HIDE), np.float32),
          np.zeros((_HIDE, _HIDE), np.float32)],
         [np.zeros((_HIDE, _HIDE), np.float32),
          np.ones((_HIDE, _HIDE), np.float32)]]))
    eh = jnp.asarray(np.concatenate(
        [np.zeros((1, _HIDE), np.float32),
         np.full((1, _HIDE), _EPS, np.float32)], axis=1))
    return npack, dpack, eps, sh, eh


def _consts():
    if _consts_cache:
        return _consts_cache[0]
    try:
        with jax.ensure_compile_time_eval():
            vals = tuple(map(jnp.asarray, _build_consts()))
        _consts_cache.append(vals)
        return vals
    except Exception:
        return _build_consts()


def _dot(a, b):
    return jax.lax.dot_general(a, b, (((1,), (0,)), ((), ())),
                               preferred_element_type=jnp.float32)


def _pass1_body(x1r, x2r, npr, dpr, wr, shr, ehr,
                fmfv_o, var_o, ps_o):
    h = _HIDE
    lane = jax.lax.broadcasted_iota(jnp.int32, (_TK, 2 * h), 1) < h
    war, wbr = wr[0:768], wr[768:1536]
    wru1r, wru2r = wr[1536:1664], wr[1664:1792]
    wc1r, wc2r = wr[1792:1920], wr[1920:2048]
    wp1r = wr[2048:2176]
    wmv1r, wmv2r = wr[2176:2304], wr[2304:2432]
    wfvr = wr[2432:2560]
    bar, bbr = wr[2560:2561], wr[2561:2562]
    bru1r, bru2r = wr[2562:2563], wr[2563:2564]
    bc1r, bp1r = wr[2564:2565], wr[2565:2566]
    bmv1r, bmv2r = wr[2566:2567], wr[2567:2568]
    bfvr = wr[2568:2569]

    def lo(z):
        return z[:, :h]

    def hi(z):
        return z[:, h:]

    def rot(z):
        return pltpu.roll(z, h, 1)

    def dlo(z):
        return jnp.where(lane, z, rot(z))

    def dhi(z):
        return jnp.where(lane, rot(z), z)

    ab1 = _dot(x1r[...], war[...]) + bar[...]
    ab2 = _dot(x2r[...], wbr[...]) + bbr[...]

    a2p = jnp.where(lane, ab1, ab2)
    comb1 = jnp.where(lane, ab1, rot(ab2))
    comb2 = jnp.where(lane, rot(ab2), ab1)
    ru1 = jax.nn.sigmoid(_dot(comb1, wru1r[...]) + bru1r[...])
    ru2 = jax.nn.sigmoid(_dot(comb2, wru2r[...]) + bru2r[...])
    ci1 = comb1 * jnp.where(lane, ru1, 1.0)
    ci2 = comb2 * jnp.where(lane, ru2, 1.0)
    cand = jnp.tanh(_dot(ci1, wc1r[...]) + _dot(ci2, wc2r[...])
                    + bc1r[...])
    u2p = jnp.where(lane, rot(ru1), ru2)
    feat = u2p * cand + (1.0 - u2p) * a2p

    h2p = jnp.maximum(_dot(feat, wp1r[...]) + bp1r[...], 0.0)
    mv1 = _dot(h2p, wmv1r[...]) + bmv1r[...]
    mv2 = _dot(h2p, wmv2r[...]) + bmv2r[...]

    eh = ehr[...]
    rec1 = 1.0 / jnp.maximum(mv1, _EPS)
    rec2 = 1.0 / jnp.maximum(mv2, _EPS)
    wf = 1.0 / (rec1 + rec2)
    meanvar = _dot(wf, wfvr[...]) + bfvr[...]

    mcvc = jnp.maximum((mv1 + mv2) * 0.5, _EPS)
    ss = jnp.sqrt(mcvc + eh)
    std2 = dhi(jnp.maximum(ss, _EPS))
    parts = dlo(mcvc) + std2 * npr[...]
    me2 = dlo(jnp.maximum(mv2, _EPS))
    rve2 = dhi(rec2)
    d = parts - me2
    dq = (d * d) * rve2
    q2 = _dot(dq, shr[...])
    wu2 = jnp.maximum(jnp.exp(-0.5 * q2), _EPS)
    s2 = (wu2 + rot(wu2)) * 0.5
    w2 = jnp.maximum(wu2 * 0.5 / s2, _EPS)
    lw = jnp.log(w2)
    t2 = dlo(rot(lw) - lw)
    idx = t2 > dpr[...]
    pn = jnp.where(idx, dhi(parts), dlo(parts))
    sw2 = w2 + rot(w2)
    wpn = w2 * pn
    fm2 = (wpn + rot(wpn)) / sw2
    df = pn - fm2
    fvt = w2 * (df * df)
    fv2 = (fvt + rot(fvt)) / sw2
    thr2 = meanvar * rot(jnp.sqrt(wf))
    cond = jnp.abs(fm2 - dlo(wf)) > dlo(thr2)
    sub = wf + eh
    fmfv = jnp.where(cond, sub,
                     jnp.where(lane, fm2, fv2))
    fvl = jnp.log(fmfv + eh)

    fmfv_o[...] = jnp.where(lane, fmfv, fvl)
    var_o[...] = hi(meanvar)
    pspack = jnp.where(lane, rot(fvl), meanvar)
    ps_o[...] = jnp.sum(pspack, axis=0, keepdims=True).reshape(1, 1, 2 * h)


def _pass2_body(fmfvr, varr, epsr, psr, qwr, qbr, wpbr, bpbr, out_o):
    h = _HIDE
    b = pl.program_id(0) // _TPB
    ps = psr[...].reshape(_NT, 2 * h)
    rows = jax.lax.broadcasted_iota(jnp.int32, (_NT, 1), 0)
    mask = (rows // _TPB) == b
    mean_row = jnp.sum(jnp.where(mask, ps, 0.0), axis=0, keepdims=True) / _N
    qs = _dot(mean_row, qwr[...]) + qbr[...]
    q0, q1 = qs[0, 0], qs[0, 1]
    mx = jnp.maximum(q0, q1)
    e0, e1 = jnp.exp(q0 - mx), jnp.exp(q1 - mx)
    w0 = e0 / (e0 + e1)
    w1 = e1 / (e0 + e1)
    fmfv = fmfvr[...]
    fvc = w0 * fmfv[:, h:] + w1 * varr[...]
    fused = epsr[...] * jnp.exp(0.5 * fvc) + fmfv[:, :h]
    out_o[...] = _dot(fused, wpbr[...]) + bpbr[...]


def _tok_spec(width):
    return pl.BlockSpec((_TK, width), lambda i: (i, 0))


def _rep_spec(shape):
    nd = len(shape)
    return pl.BlockSpec(shape, lambda i, _n=nd: (0,) * _n)


def _run(x1, x2, params, interpret=False):
    p = params
    npack, dpack, eps, sh, eh = _consts()
    cat = jnp.concatenate
    h = _HIDE
    z64 = jnp.zeros((h, h), jnp.float32)
    wa = cat([p["g1_p1_w"], p["g2_p2_w"]], 1)
    ba = cat([p["g1_p1_b"], p["g2_p2_b"]])[None]
    wb = cat([p["g1_p2_w"], p["g2_p1_w"]], 1)
    bb = cat([p["g1_p2_b"], p["g2_p1_b"]])[None]
    wru1 = cat([p["g1_r_w"], p["g1_u_w"]], 1)
    bru1 = cat([p["g1_r_b"], p["g1_u_b"]])[None]
    wru2 = cat([p["g2_r_w"], p["g2_u_w"]], 1)
    bru2 = cat([p["g2_r_b"], p["g2_u_b"]])[None]
    z128 = jnp.zeros((2 * h, h), jnp.float32)
    wc1 = cat([p["g1_c_w"], z128], 1)
    wc2 = cat([z128, p["g2_c_w"]], 1)
    bc12 = cat([p["g1_c_b"], p["g2_c_b"]])[None]
    wpd = cat([cat([p["proj1_w"], z64], 1),
               cat([z64, p["proj2_w"]], 1)], 0)
    bpd = cat([p["proj1_b"], p["proj2_b"]])[None]
    zr = jnp.zeros((h, 2 * h), jnp.float32)
    wmv1 = cat([cat([p["fcmean1_w"], p["fcvar1_w"]], 1), zr], 0)
    bmv1 = cat([p["fcmean1_b"], p["fcvar1_b"]])[None]
    wmv2 = cat([zr, cat([p["fcmean2_w"], p["fcvar2_w"]], 1)], 0)
    bmv2 = cat([p["fcmean2_b"], p["fcvar2_b"]])[None]
    wfv = cat([cat([p["fuse_mean_w"], z64], 1),
               cat([z64, p["fuse_var_w"]], 1)], 0)
    bfv = cat([p["fuse_mean_b"], p["fuse_var_b"]])[None]
    w_all = cat([wa, wb, wru1, wru2, wc1, wc2, wpd, wmv1, wmv2, wfv,
                 ba, bb, bru1, bru2, bc12, bpd, bmv1, bmv2, bfv,
                 jnp.zeros((7, 2 * h), jnp.float32)], 0)

    f32 = jnp.float32
    fmfv, var, ps = pl.pallas_call(
        _pass1_body,
        grid=(_NT,),
        in_specs=[
            _tok_spec(_INP), _tok_spec(_INP),
            _tok_spec(2 * h), _tok_spec(2 * h),
            _rep_spec((2576, 2 * h)),
            _rep_spec((2 * h, 2 * h)), _rep_spec((1, 2 * h)),
        ],
        out_specs=[
            _tok_spec(2 * h), _tok_spec(h),
            pl.BlockSpec((1, 1, 2 * h), lambda i: (i, 0, 0)),
        ],
        out_shape=[
            jax.ShapeDtypeStruct((_T, 2 * h), f32),
            jax.ShapeDtypeStruct((_T, h), f32),
            jax.ShapeDtypeStruct((_NT, 1, 2 * h), f32),
        ],
        interpret=interpret,
    )(x1, x2, npack, dpack, w_all, sh, eh)

    out = pl.pallas_call(
        _pass2_body,
        grid=(_NT,),
        in_specs=[
            _tok_spec(2 * h), _tok_spec(h), _tok_spec(h),
            _rep_spec((_NT, 1, 2 * h)),
            _rep_spec((2 * h, 2)), _rep_spec((1, 2)),
            _rep_spec((h, _INP)), _rep_spec((1, _INP)),
        ],
        out_specs=[_tok_spec(_INP)],
        out_shape=[jax.ShapeDtypeStruct((_T, _INP), f32)],
        interpret=interpret,
    )(fmfv, var, eps, ps, p["qe_w"], p["qe_b"][None],
      p["proj_back_w"], p["proj_back_b"][None])[0]
    return out


def kernel(feature_1, feature_2, params):
    x1 = feature_1.reshape(_T, _INP)
    x2 = feature_2.reshape(_T, _INP)
    return _run(x1, x2, params).reshape(_B, _N, _INP)


# speedup vs baseline: 1.1330x; 1.1330x over previous
import jax
import jax.numpy as jnp
import numpy as np
from jax.experimental import pallas as pl
from jax.experimental.pallas import tpu as pltpu

_B, _N, _INP, _HIDE, _P = 4, 4096, 768, 64, 2
_EPS = 1e-6
_TK = 2048
_T = _B * _N
_NT = _T // _TK
_TPB = _N // _TK

_consts_cache = []


def _build_consts():
    key = jax.random.key(42)
    noise = jax.random.normal(
        jax.random.fold_in(key, 0), (_P, _B, _N, _HIDE), jnp.float32)
    g = jax.random.gumbel(
        jax.random.fold_in(key, 1), (_P, _B * _N * _HIDE, _P), jnp.float32)
    d = (g[..., 0] - g[..., 1]).reshape(_P, _T, _HIDE)
    eps = jax.random.normal(
        jax.random.fold_in(key, 99), (_B, _N, _HIDE),
        jnp.float32).reshape(_T, _HIDE)
    npack = jnp.concatenate(
        [noise[0].reshape(_T, _HIDE), noise[1].reshape(_T, _HIDE)], axis=1)
    dpack = jnp.concatenate([d[0], d[1]], axis=1)
    sh = jnp.asarray(np.block(
        [[np.ones((_HIDE, _HIDE), np.float32),
          np.zeros((_HIDE, _HIDE), np.float32)],
         [np.zeros((_HIDE, _HIDE), np.float32),
          np.ones((_HIDE, _HIDE), np.float32)]]))
    eh = jnp.asarray(np.concatenate(
        [np.zeros((1, _HIDE), np.float32),
         np.full((1, _HIDE), _EPS, np.float32)], axis=1))
    sg = jnp.asarray(np.concatenate(
        [np.ones((1, _HIDE), np.float32),
         np.full((1, _HIDE), -1.0, np.float32)], axis=1))
    return npack, dpack, eps, sh, jnp.concatenate([eh, sg], axis=0)


def _consts():
    if _consts_cache:
        return _consts_cache[0]
    try:
        with jax.ensure_compile_time_eval():
            vals = tuple(map(jnp.asarray, _build_consts()))
        _consts_cache.append(vals)
        return vals
    except Exception:
        return _build_consts()


def _dot(a, b):
    return jax.lax.dot_general(a, b, (((1,), (0,)), ((), ())),
                               preferred_element_type=jnp.float32)


def _pass1_body(x1r, x2r, npr, dpr, shr, ehr,
                g1p1w, g1p1b, g1p2w, g1p2b, g1rw, g1rb, g1uw, g1ub,
                g1cw, g1cb,
                g2p1w, g2p1b, g2p2w, g2p2b, g2rw, g2rb, g2uw, g2ub,
                g2cw, g2cb,
                p1w, p1b, p2w, p2b,
                m1w, m1b, v1w, v1b, m2w, m2b, v2w, v2b,
                fmw, fmb, fvw, fvb,
                fmfv_o, var_o, ps_o, wsc):
    h = _HIDE
    lane = jax.lax.broadcasted_iota(jnp.int32, (_TK, 2 * h), 1) < h

    @pl.when(pl.program_id(0) == 0)
    def _pack():
        wsc[...] = jnp.zeros((2576, 2 * h), jnp.float32)
        wsc[0:768, 0:h] = g1p1w[...]
        wsc[0:768, h:] = g2p2w[...]
        wsc[768:1536, 0:h] = g1p2w[...]
        wsc[768:1536, h:] = g2p1w[...]
        wsc[1536:1664, 0:h] = g1rw[...]
        wsc[1536:1664, h:] = g1uw[...]
        wsc[1664:1792, 0:h] = g2rw[...]
        wsc[1664:1792, h:] = g2uw[...]
        wsc[1792:1920, 0:h] = g1cw[...]
        wsc[1920:2048, h:] = g2cw[...]
        wsc[2048:2112, 0:h] = p1w[...]
        wsc[2112:2176, h:] = p2w[...]
        wsc[2176:2240, 0:h] = m1w[...]
        wsc[2176:2240, h:] = v1w[...]
        wsc[2368:2432, 0:h] = m2w[...]
        wsc[2368:2432, h:] = v2w[...]
        wsc[2432:2496, 0:h] = fmw[...]
        wsc[2496:2560, h:] = fvw[...]
        wsc[2560:2561, 0:h] = g1p1b[...]
        wsc[2560:2561, h:] = g2p2b[...]
        wsc[2561:2562, 0:h] = g1p2b[...]
        wsc[2561:2562, h:] = g2p1b[...]
        wsc[2562:2563, 0:h] = g1rb[...]
        wsc[2562:2563, h:] = g1ub[...]
        wsc[2563:2564, 0:h] = g2rb[...]
        wsc[2563:2564, h:] = g2ub[...]
        wsc[2564:2565, 0:h] = g1cb[...]
        wsc[2564:2565, h:] = g2cb[...]
        wsc[2565:2566, 0:h] = p1b[...]
        wsc[2565:2566, h:] = p2b[...]
        wsc[2566:2567, 0:h] = m1b[...]
        wsc[2566:2567, h:] = v1b[...]
        wsc[2567:2568, 0:h] = m2b[...]
        wsc[2567:2568, h:] = v2b[...]
        wsc[2568:2569, 0:h] = fmb[...]
        wsc[2568:2569, h:] = fvb[...]

    war, wbr = wsc[0:768], wsc[768:1536]
    wru1r, wru2r = wsc[1536:1664], wsc[1664:1792]
    wc1r, wc2r = wsc[1792:1920], wsc[1920:2048]
    wp1r = wsc[2048:2176]
    wmv1r, wmv2r = wsc[2176:2304], wsc[2304:2432]
    wfvr = wsc[2432:2560]
    bar, bbr = wsc[2560:2561], wsc[2561:2562]
    bru1r, bru2r = wsc[2562:2563], wsc[2563:2564]
    bc1r, bp1r = wsc[2564:2565], wsc[2565:2566]
    bmv1r, bmv2r = wsc[2566:2567], wsc[2567:2568]
    bfvr = wsc[2568:2569]

    def lo(z):
        return z[:, :h]

    def hi(z):
        return z[:, h:]

    def rot(z):
        return pltpu.roll(z, h, 1)

    def dlo(z):
        return jnp.where(lane, z, rot(z))

    def dhi(z):
        return jnp.where(lane, rot(z), z)

    ab1 = _dot(x1r[...], war[...]) + bar[...]
    ab2 = _dot(x2r[...], wbr[...]) + bbr[...]

    a2p = jnp.where(lane, ab1, ab2)
    comb1 = jnp.where(lane, ab1, rot(ab2))
    comb2 = jnp.where(lane, rot(ab2), ab1)
    ru1 = jax.nn.sigmoid(_dot(comb1, wru1r[...]) + bru1r[...])
    ru2 = jax.nn.sigmoid(_dot(comb2, wru2r[...]) + bru2r[...])
    ci1 = comb1 * jnp.where(lane, ru1, 1.0)
    ci2 = comb2 * jnp.where(lane, ru2, 1.0)
    cand = jnp.tanh(_dot(ci1, wc1r[...]) + _dot(ci2, wc2r[...])
                    + bc1r[...])
    u2p = jnp.where(lane, rot(ru1), ru2)
    feat = u2p * cand + (1.0 - u2p) * a2p

    h2p = jnp.maximum(_dot(feat, wp1r[...]) + bp1r[...], 0.0)
    mv1 = _dot(h2p, wmv1r[...]) + bmv1r[...]
    mv2 = _dot(h2p, wmv2r[...]) + bmv2r[...]

    eh = ehr[0:1]
    sg = ehr[1:2]
    rec1 = 1.0 / jnp.maximum(mv1, _EPS)
    rec2 = 1.0 / jnp.maximum(mv2, _EPS)
    wf = 1.0 / (rec1 + rec2)
    meanvar = _dot(wf, wfvr[...]) + bfvr[...]

    mcvc = jnp.maximum((mv1 + mv2) * 0.5, _EPS)
    ss = jnp.sqrt(mcvc + eh)
    std2 = dhi(jnp.maximum(ss, _EPS))
    parts = dlo(mcvc) + std2 * npr[...]
    me2 = dlo(jnp.maximum(mv2, _EPS))
    rve2 = dhi(rec2)
    d = parts - me2
    dq = (d * d) * rve2
    q2 = _dot(dq, shr[...])
    wu2 = jnp.maximum(jnp.exp(-0.5 * q2), _EPS)
    s2 = (wu2 + rot(wu2)) * 0.5
    w2 = jnp.maximum(wu2 * 0.5 / s2, _EPS)
    lw = jnp.log(w2)
    t2 = (rot(lw) - lw) * sg
    idx = t2 > dpr[...]
    pn = jnp.where(idx, dhi(parts), dlo(parts))
    sw2 = w2 + rot(w2)
    wpn = w2 * pn
    fm2 = (wpn + rot(wpn)) / sw2
    df = pn - fm2
    fvt = w2 * (df * df)
    fv2 = (fvt + rot(fvt)) / sw2
    thr2 = meanvar * rot(jnp.sqrt(wf))
    cond = dlo(jnp.abs(fm2 - wf) - thr2) > 0.0
    sub = wf + eh
    fmfv = jnp.where(cond, sub,
                     jnp.where(lane, fm2, fv2))
    fvl = jnp.log(fmfv + eh)

    fmfv_o[...] = jnp.where(lane, fmfv, fvl)
    var_o[...] = hi(meanvar)
    pspack = jnp.where(lane, rot(fvl), meanvar)
    ps_o[...] = jnp.sum(pspack, axis=0, keepdims=True).reshape(1, 1, 2 * h)


def _pass2_body(fmfvr, varr, epsr, psr, qwr, qbr, wpbr, bpbr, out_o):
    h = _HIDE
    b = pl.program_id(0) // _TPB
    ps = psr[...].reshape(_NT, 2 * h)
    rows = jax.lax.broadcasted_iota(jnp.int32, (_NT, 1), 0)
    mask = (rows // _TPB) == b
    mean_row = jnp.sum(jnp.where(mask, ps, 0.0), axis=0, keepdims=True) / _N
    qs = _dot(mean_row, qwr[...]) + qbr[...]
    q0, q1 = qs[0, 0], qs[0, 1]
    mx = jnp.maximum(q0, q1)
    e0, e1 = jnp.exp(q0 - mx), jnp.exp(q1 - mx)
    w0 = e0 / (e0 + e1)
    w1 = e1 / (e0 + e1)
    fmfv = fmfvr[...]
    fvc = w0 * fmfv[:, h:] + w1 * varr[...]
    fused = epsr[...] * jnp.exp(0.5 * fvc) + fmfv[:, :h]
    out_o[...] = _dot(fused, wpbr[...]) + bpbr[...]


def _tok_spec(width):
    return pl.BlockSpec((_TK, width), lambda i: (i, 0))


def _rep_spec(shape):
    nd = len(shape)
    return pl.BlockSpec(shape, lambda i, _n=nd: (0,) * _n)


def _run(x1, x2, params, interpret=False):
    p = params
    npack, dpack, eps, sh, eh = _consts()
    h = _HIDE
    f32 = jnp.float32
    raw_names = ("g1_p1", "g1_p2", "g1_r", "g1_u", "g1_c",
                 "g2_p1", "g2_p2", "g2_r", "g2_u", "g2_c",
                 "proj1", "proj2",
                 "fcmean1", "fcvar1", "fcmean2", "fcvar2",
                 "fuse_mean", "fuse_var")
    raw = {}
    for nm in raw_names:
        raw[nm + "_w"] = p[nm + "_w"]
        raw[nm + "_b"] = p[nm + "_b"][None]
    raw_args = []
    raw_specs = []
    ordered = ["g1_p1", "g1_p2", "g1_r", "g1_u", "g1_c",
               "g2_p1", "g2_p2", "g2_r", "g2_u", "g2_c",
               "proj1", "proj2",
               "fcmean1", "fcvar1", "fcmean2", "fcvar2",
               "fuse_mean", "fuse_var"]
    for nm in ordered:
        w = raw[nm + "_w"]
        raw_args += [w, raw[nm + "_b"]]
        raw_specs += [_rep_spec(w.shape), _rep_spec((1, h))]

    fmfv, var, ps = pl.pallas_call(
        _pass1_body,
        grid=(_NT,),
        in_specs=[
            _tok_spec(_INP), _tok_spec(_INP),
            _tok_spec(2 * h), _tok_spec(2 * h),
            _rep_spec((2 * h, 2 * h)), _rep_spec((2, 2 * h)),
        ] + raw_specs,
        out_specs=[
            _tok_spec(2 * h), _tok_spec(h),
            pl.BlockSpec((1, 1, 2 * h), lambda i: (i, 0, 0)),
        ],
        out_shape=[
            jax.ShapeDtypeStruct((_T, 2 * h), f32),
            jax.ShapeDtypeStruct((_T, h), f32),
            jax.ShapeDtypeStruct((_NT, 1, 2 * h), f32),
        ],
        scratch_shapes=[pltpu.VMEM((2576, 2 * h), f32)],
        interpret=interpret,
    )(x1, x2, npack, dpack, sh, eh, *raw_args)

    out = pl.pallas_call(
        _pass2_body,
        grid=(_NT,),
        in_specs=[
            _tok_spec(2 * h), _tok_spec(h), _tok_spec(h),
            _rep_spec((_NT, 1, 2 * h)),
            _rep_spec((2 * h, 2)), _rep_spec((1, 2)),
            _rep_spec((h, _INP)), _rep_spec((1, _INP)),
        ],
        out_specs=[_tok_spec(_INP)],
        out_shape=[jax.ShapeDtypeStruct((_T, _INP), f32)],
        interpret=interpret,
    )(fmfv, var, eps, ps, p["qe_w"], p["qe_b"][None],
      p["proj_back_w"], p["proj_back_b"][None])[0]
    return out


def kernel(feature_1, feature_2, params):
    x1 = feature_1.reshape(_T, _INP)
    x2 = feature_2.reshape(_T, _INP)
    return _run(x1, x2, params).reshape(_B, _N, _INP)
